# Initial kernel scaffold; baseline (speedup 1.0000x reference)
import jax
import jax.numpy as jnp
from jax.experimental import pallas as pl


def kernel(x, edge_index, W1, b1, W_mu, b_mu, W_ls, b_ls):
    raise NotImplementedError("write your pallas kernel here")


# speedup vs baseline: 9.7548x; 9.7548x over previous
import dataclasses
import functools

import jax
import jax.numpy as jnp
from jax import lax
from jax.experimental import pallas as pl
from jax.experimental.pallas import tpu as pltpu
from jax.experimental.pallas import tpu_sc as plsc

N_NODES = 10000
N_PAD = 10240
D_PAD = 128
N_EDGES = 320000
NC = 2
NS = 16
NW = NC * NS
CHUNK = 128
N_CHUNKS = N_EDGES // CHUNK


def _vector_mesh():
    return plsc.VectorSubcoreMesh(core_axis_name="c", subcore_axis_name="s")


def _my_chunks(wid):
    full = N_CHUNKS // NW
    extra = N_CHUNKS % NW
    return full + jnp.where(wid < extra, 1, 0)


def _sc_degree(dst):
    @functools.partial(
        pl.kernel,
        out_type=jax.ShapeDtypeStruct((NC, N_PAD), jnp.float32),
        mesh=_vector_mesh(),
        scratch_types=[
            pltpu.VMEM_SHARED((N_PAD,), jnp.float32),
            pltpu.VMEM((CHUNK,), jnp.int32),
            pltpu.VMEM((CHUNK,), jnp.float32),
        ],
    )
    def k(dst_hbm, zeros_hbm, ones_hbm, out_hbm, acc_sh, idx_v, ones_v):
        cid = lax.axis_index("c")
        sid = lax.axis_index("s")
        wid = sid * NC + cid
        rpt = N_PAD // NS
        r0 = sid * rpt
        pltpu.sync_copy(ones_hbm, ones_v)
        pltpu.sync_copy(zeros_hbm.at[pl.ds(r0, rpt)], acc_sh.at[pl.ds(r0, rpt)])
        plsc.subcore_barrier()

        @pl.loop(0, _my_chunks(wid))
        def _(t):
            base = (wid + t * NW) * CHUNK
            pltpu.sync_copy(dst_hbm.at[pl.ds(base, CHUNK)], idx_v)
            pltpu.sync_copy(ones_v, acc_sh.at[idx_v], add=True)

        plsc.subcore_barrier()
        pltpu.sync_copy(acc_sh.at[pl.ds(r0, rpt)],
                        out_hbm.at[cid].at[pl.ds(r0, rpt)])

    return k(dst, jnp.zeros((N_PAD,), jnp.float32),
             jnp.ones((CHUNK,), jnp.float32))


def _sc_aggregate(h, src, dst, zeros2d):
    n, d = h.shape
    assert d == D_PAD

    @functools.partial(
        pl.kernel,
        out_type=jax.ShapeDtypeStruct((NC, N_PAD, d), jnp.float32),
        mesh=_vector_mesh(),
        scratch_types=[
            pltpu.VMEM_SHARED((N_PAD, d), jnp.float32),
            pltpu.VMEM((CHUNK,), jnp.int32),
            pltpu.VMEM((CHUNK,), jnp.int32),
            pltpu.VMEM((CHUNK, d), jnp.float32),
        ],
    )
    def k(h_hbm, src_hbm, dst_hbm, z_hbm, out_hbm, acc_sh, src_v, dst_v, rows_v):
        cid = lax.axis_index("c")
        sid = lax.axis_index("s")
        wid = sid * NC + cid
        rpt = N_PAD // NS
        r0 = sid * rpt
        pltpu.sync_copy(z_hbm.at[pl.ds(r0, rpt)], acc_sh.at[pl.ds(r0, rpt)])
        plsc.subcore_barrier()

        @pl.loop(0, _my_chunks(wid))
        def _(t):
            base = (wid + t * NW) * CHUNK
            pltpu.sync_copy(src_hbm.at[pl.ds(base, CHUNK)], src_v)
            pltpu.sync_copy(dst_hbm.at[pl.ds(base, CHUNK)], dst_v)
            pltpu.sync_copy(h_hbm.at[src_v], rows_v)
            pltpu.sync_copy(rows_v, acc_sh.at[dst_v], add=True)

        plsc.subcore_barrier()
        pltpu.sync_copy(acc_sh.at[pl.ds(r0, rpt)],
                        out_hbm.at[cid].at[pl.ds(r0, rpt)])

    return k(h, src, dst, zeros2d)


def _sc_edge_dot(z, src, dst, d_real):
    n, d = z.shape
    assert d == D_PAD

    cp = pltpu.CompilerParams()
    if "needs_layout_passes" in pltpu.CompilerParams.__dataclass_fields__:
        cp = dataclasses.replace(cp, needs_layout_passes=False)

    @functools.partial(
        pl.kernel,
        out_type=jax.ShapeDtypeStruct((N_EDGES,), jnp.float32),
        mesh=_vector_mesh(),
        compiler_params=cp,
        scratch_types=[
            pltpu.VMEM((CHUNK,), jnp.int32),
            pltpu.VMEM((CHUNK,), jnp.int32),
            pltpu.VMEM((CHUNK, d), jnp.float32),
            pltpu.VMEM((CHUNK, d), jnp.float32),
            pltpu.VMEM((CHUNK,), jnp.float32),
        ],
    )
    def k(z_hbm, src_hbm, dst_hbm, out_hbm, src_v, dst_v, a_v, b_v, o_v):
        cid = lax.axis_index("c")
        sid = lax.axis_index("s")
        wid = sid * NC + cid

        @pl.loop(0, _my_chunks(wid))
        def _(t):
            base = (wid + t * NW) * CHUNK
            pltpu.sync_copy(src_hbm.at[pl.ds(base, CHUNK)], src_v)
            pltpu.sync_copy(dst_hbm.at[pl.ds(base, CHUNK)], dst_v)
            pltpu.sync_copy(z_hbm.at[src_v], a_v)
            pltpu.sync_copy(z_hbm.at[dst_v], b_v)

            @pl.loop(0, CHUNK, step=16)
            def _(e):
                rows = e + lax.iota(jnp.int32, 16)
                acc = jnp.zeros((16,), jnp.float32)
                for col in range(d_real):
                    cols = jnp.full((16,), col, jnp.int32)
                    va = plsc.load_gather(a_v, [rows, cols])
                    vb = plsc.load_gather(b_v, [rows, cols])
                    acc = acc + va * vb
                o_v[pl.ds(e, 16)] = acc

            pltpu.sync_copy(o_v, out_hbm.at[pl.ds(base, CHUNK)])

    return k(z, src, dst)


def _tc_stage1(x, w1, deg_parts):
    n = x.shape[0]
    h = w1.shape[1]

    def body(x_ref, w_ref, deg_ref, h0p_ref, dis_ref):
        deg = deg_ref[0, :n] + deg_ref[1, :n] + 1.0
        dis = lax.rsqrt(deg)
        h0 = jnp.dot(x_ref[...], w_ref[...], preferred_element_type=jnp.float32)
        h0p_ref[:, :h] = h0 * dis[:, None]
        h0p_ref[:, h:] = jnp.zeros((n, D_PAD - h), jnp.float32)
        dis_ref[...] = dis[:, None]

    return pl.pallas_call(
        body,
        out_shape=(jax.ShapeDtypeStruct((n, D_PAD), jnp.float32),
                   jax.ShapeDtypeStruct((n, 1), jnp.float32)),
    )(x, w1, deg_parts)


def _tc_stage2(acc_parts, h0p, dis, b1):
    n = h0p.shape[0]
    h = b1.shape[1]

    def body(acc_ref, h0p_ref, dis_ref, b1_ref, h1p_ref):
        s = acc_ref[0, :n, :h] + acc_ref[1, :n, :h] + h0p_ref[:, :h]
        h1 = jnp.maximum(s * dis_ref[...] + b1_ref[...], 0.0)
        h1p_ref[:, :h] = h1 * dis_ref[...]
        h1p_ref[:, h:] = jnp.zeros((n, D_PAD - h), jnp.float32)

    return pl.pallas_call(
        body,
        out_shape=jax.ShapeDtypeStruct((n, D_PAD), jnp.float32),
    )(acc_parts, h0p, dis, b1)


def _tc_stage3(acc_parts, h1p, dis, w_mu, b_mu, w_ls, b_ls):
    n = h1p.shape[0]
    h = w_mu.shape[0]
    o = w_mu.shape[1]

    def body(acc_ref, h1p_ref, dis_ref, wm_ref, bm_ref, wl_ref, bl_ref,
             mu_ref, ls_ref, mup_ref):
        g = (acc_ref[0, :n, :h] + acc_ref[1, :n, :h] + h1p_ref[:, :h]) \
            * dis_ref[...]
        mu = jnp.dot(g, wm_ref[...],
                     preferred_element_type=jnp.float32) + bm_ref[...]
        mu_ref[...] = mu
        ls_ref[...] = jnp.dot(g, wl_ref[...],
                              preferred_element_type=jnp.float32) + bl_ref[...]
        mup_ref[:, :o] = mu
        mup_ref[:, o:] = jnp.zeros((n, D_PAD - o), jnp.float32)

    return pl.pallas_call(
        body,
        out_shape=(jax.ShapeDtypeStruct((n, o), jnp.float32),
                   jax.ShapeDtypeStruct((n, o), jnp.float32),
                   jax.ShapeDtypeStruct((n, D_PAD), jnp.float32)),
    )(acc_parts, h1p, dis, w_mu, b_mu, w_ls, b_ls)


def kernel(x, edge_index, W1, b1, W_mu, b_mu, W_ls, b_ls):
    src = edge_index[0].astype(jnp.int32)
    dst = edge_index[1].astype(jnp.int32)
    zeros2d = jnp.zeros((N_PAD, D_PAD), jnp.float32)

    deg_parts = _sc_degree(dst)
    h0p, dis = _tc_stage1(x, W1, deg_parts)
    acc1 = _sc_aggregate(h0p, src, dst, zeros2d)
    h1p = _tc_stage2(acc1, h0p, dis, b1.reshape(1, -1))
    acc2 = _sc_aggregate(h1p, src, dst, zeros2d)
    mu, logstd, mu_pad = _tc_stage3(acc2, h1p, dis, W_mu, b_mu.reshape(1, -1),
                                    W_ls, b_ls.reshape(1, -1))
    adj_pred = _sc_edge_dot(mu_pad, src, dst, W_mu.shape[1])
    return adj_pred, mu, logstd
